# Initial kernel scaffold; baseline (speedup 1.0000x reference)
import jax
import jax.numpy as jnp
from jax.experimental import pallas as pl


def kernel(x, edge_index, batch, W0, b0, Wc, W1, b1, W2, b2):
    raise NotImplementedError("write your pallas kernel here")


# speedup vs baseline: 5.6822x; 5.6822x over previous
import functools

import jax
import jax.numpy as jnp
from jax import lax
from jax.experimental import pallas as pl
from jax.experimental.pallas import tpu as pltpu
from jax.experimental.pallas import tpu_sc as plsc

F32 = jnp.float32
I32 = jnp.int32

NC = 2
NS = 16
ECH = 128


def _mesh():
    return plsc.VectorSubcoreMesh(
        core_axis_name="c", subcore_axis_name="s", num_cores=NC, num_subcores=NS
    )


def _make_sc_deg(n_pad, d, chunks_per_tile):
    cpc = chunks_per_tile * NS
    rpt = n_pad // NS

    def body(dst0, dst1, ones_h, zrows, out, acc, idxs, onesv, sem):
        cid = lax.axis_index("c")
        sid = lax.axis_index("s")
        r0 = sid * rpt
        ch0 = cid * cpc + sid * chunks_per_tile
        pltpu.sync_copy(ones_h, onesv)
        for a, dst in ((0, dst0), (1, dst1)):
            pltpu.sync_copy(zrows, acc.at[pl.ds(r0, rpt)])
            plsc.subcore_barrier()

            def group(gi, _):
                pltpu.sync_copy(dst.at[pl.ds(ch0 + gi * 8, 8)], idxs)
                for j in range(8):
                    pltpu.sync_copy(onesv, acc.at[idxs.at[j]], add=True)
                return 0

            lax.fori_loop(0, chunks_per_tile // 8, group, 0)
            plsc.subcore_barrier()
            pltpu.sync_copy(acc.at[pl.ds(r0, rpt)], out.at[a, cid, pl.ds(r0, rpt)])
            plsc.subcore_barrier()

    return pl.kernel(
        body,
        out_type=jax.ShapeDtypeStruct((2, NC, n_pad, d), F32),
        mesh=_mesh(),
        scratch_types=[
            pltpu.VMEM_SHARED((n_pad, d), F32),
            pltpu.VMEM((8, ECH), I32),
            pltpu.VMEM((ECH, d), F32),
            pltpu.SemaphoreType.DMA,
        ],
    )


def _make_sc_conv(n_pad, d, chunks_per_tile):
    cpc = chunks_per_tile * NS
    rpt = n_pad // NS
    zch = rpt // ECH

    grp = 8
    ngrp = chunks_per_tile // grp

    def body(g, src0, dst0, src1, dst1, zrows, out, acc, idxs, idxd, rows, sems):
        cid = lax.axis_index("c")
        sid = lax.axis_index("s")
        r0 = sid * rpt
        ch0 = cid * cpc + sid * chunks_per_tile
        for a, src, dst in ((0, src0, dst0), (1, src1, dst1)):
            ga = g.at[a]
            pltpu.sync_copy(zrows, acc.at[pl.ds(r0, rpt)])
            plsc.subcore_barrier()

            def group(gi, _):
                gch = ch0 + gi * grp
                pltpu.sync_copy(src.at[pl.ds(gch, grp)], idxs)
                pltpu.sync_copy(dst.at[pl.ds(gch, grp)], idxd)
                pltpu.async_copy(ga.at[idxs.at[0]], rows.at[0], sems.at[0])
                for j in range(grp):
                    p = j % 2
                    pltpu.make_async_copy(
                        ga.at[idxs.at[j]], rows.at[p], sems.at[p]
                    ).wait()
                    if j + 1 < grp:
                        pltpu.async_copy(
                            ga.at[idxs.at[j + 1]], rows.at[1 - p], sems.at[1 - p]
                        )
                    pltpu.sync_copy(rows.at[p], acc.at[idxd.at[j]], add=True)
                return 0

            lax.fori_loop(0, ngrp, group, 0)
            plsc.subcore_barrier()
            for k in range(zch):
                pltpu.sync_copy(
                    acc.at[pl.ds(r0 + k * ECH, ECH)],
                    out.at[a, cid, pl.ds(r0 + k * ECH, ECH)],
                )
            plsc.subcore_barrier()

    return pl.kernel(
        body,
        out_type=jax.ShapeDtypeStruct((2, NC, n_pad, d), F32),
        mesh=_mesh(),
        scratch_types=[
            pltpu.VMEM_SHARED((n_pad, d), F32),
            pltpu.VMEM((grp, ECH), I32),
            pltpu.VMEM((grp, ECH), I32),
            pltpu.VMEM((2, ECH, d), F32),
            pltpu.SemaphoreType.DMA((2,)),
        ],
    )


def _make_sc_pool(n_pad, d, b_pad):
    nch = n_pad // ECH
    cpt = -(-nch // (NC * NS))

    def body(h3, batch2d, zrows, out, acc, idxb, rows, sem):
        cid = lax.axis_index("c")
        sid = lax.axis_index("s")
        wid = cid * NS + sid
        pltpu.sync_copy(batch2d, idxb)

        @pl.when(sid == 0)
        def _():
            for a in range(2):
                pltpu.sync_copy(zrows.at[pl.ds(0, b_pad)], acc.at[a])

        plsc.subcore_barrier()
        for a in range(2):
            for k in range(cpt):
                ch = k * NC * NS + wid

                @pl.when(ch < nch)
                def _():
                    pltpu.sync_copy(h3.at[a, pl.ds(ch * ECH, ECH)], rows)
                    pltpu.sync_copy(rows, acc.at[a].at[idxb.at[ch]], add=True)

        plsc.subcore_barrier()

        @pl.when(sid == 0)
        def _():
            for a in range(2):
                pltpu.sync_copy(acc.at[a], out.at[a, cid])

    return pl.kernel(
        body,
        out_type=jax.ShapeDtypeStruct((2, NC, b_pad, d), F32),
        mesh=_mesh(),
        scratch_types=[
            pltpu.VMEM_SHARED((2, b_pad, d), F32),
            pltpu.VMEM((nch, ECH), I32),
            pltpu.VMEM((ECH, d), F32),
            pltpu.SemaphoreType.DMA,
        ],
    )


def _tc_h(xp, w0, b0r, n_pad, d, rb):
    def body(x_ref, w_ref, b_ref, o_ref):
        o_ref[...] = jax.nn.relu(
            jnp.dot(x_ref[...], w_ref[...], preferred_element_type=F32) + b_ref[...]
        )

    return pl.pallas_call(
        body,
        grid=(n_pad // rb,),
        in_specs=[
            pl.BlockSpec((rb, d), lambda i: (i, 0)),
            pl.BlockSpec((d, d), lambda i: (0, 0)),
            pl.BlockSpec((1, d), lambda i: (0, 0)),
        ],
        out_specs=pl.BlockSpec((rb, d), lambda i: (i, 0)),
        out_shape=jax.ShapeDtypeStruct((n_pad, d), F32),
    )(xp, w0, b0r)


def _tc_dinv(deg, n_pad, d, rb):
    nrb = n_pad // rb

    def body(a_ref, b_ref, o_ref):
        dsum = a_ref[0, 0, :, 0] + b_ref[0, 0, :, 0] + 1.0
        dv = lax.rsqrt(dsum)
        o_ref[...] = jnp.broadcast_to(dv.reshape(1, rb, 1), (1, rb, d))

    return pl.pallas_call(
        body,
        grid=(2, nrb),
        in_specs=[
            pl.BlockSpec((1, 1, rb, d), lambda a, r: (a, 0, r, 0)),
            pl.BlockSpec((1, 1, rb, d), lambda a, r: (a, 1, r, 0)),
        ],
        out_specs=pl.BlockSpec((1, rb, d), lambda a, r: (a, r, 0)),
        out_shape=jax.ShapeDtypeStruct((2, n_pad, d), F32),
    )(deg, deg)


def _tc_g1(h, wl, dinv, n_pad, d, rb):
    def body(h_ref, w_ref, v_ref, o_ref):
        o_ref[0] = v_ref[0] * jnp.dot(
            h_ref[...], w_ref[0], preferred_element_type=F32
        )

    return pl.pallas_call(
        body,
        grid=(2, n_pad // rb),
        in_specs=[
            pl.BlockSpec((rb, d), lambda a, r: (r, 0)),
            pl.BlockSpec((1, d, d), lambda a, r: (a, 0, 0)),
            pl.BlockSpec((1, rb, d), lambda a, r: (a, r, 0)),
        ],
        out_specs=pl.BlockSpec((1, rb, d), lambda a, r: (a, r, 0)),
        out_shape=jax.ShapeDtypeStruct((2, n_pad, d), F32),
    )(h, wl, dinv)


def _tc_step(p, g, dinv, wl, n_pad, d, rb):
    def body(p0_ref, p1_ref, g_ref, v_ref, w_ref, o_ref):
        v = v_ref[0]
        hn = jax.nn.relu(v * (p0_ref[0, 0] + p1_ref[0, 0] + g_ref[0]))
        o_ref[0] = v * jnp.dot(hn, w_ref[0], preferred_element_type=F32)

    bs = pl.BlockSpec((1, rb, d), lambda a, r: (a, r, 0))
    return pl.pallas_call(
        body,
        grid=(2, n_pad // rb),
        in_specs=[
            pl.BlockSpec((1, 1, rb, d), lambda a, r: (a, 0, r, 0)),
            pl.BlockSpec((1, 1, rb, d), lambda a, r: (a, 1, r, 0)),
            bs,
            bs,
            pl.BlockSpec((1, d, d), lambda a, r: (a, 0, 0)),
        ],
        out_specs=bs,
        out_shape=jax.ShapeDtypeStruct((2, n_pad, d), F32),
    )(p, p, g, dinv, wl)


def _tc_final(p, g, dinv, n_pad, d, rb):
    def body(p0_ref, p1_ref, g_ref, v_ref, o_ref):
        o_ref[0] = jax.nn.relu(v_ref[0] * (p0_ref[0, 0] + p1_ref[0, 0] + g_ref[0]))

    bs = pl.BlockSpec((1, rb, d), lambda a, r: (a, r, 0))
    return pl.pallas_call(
        body,
        grid=(2, n_pad // rb),
        in_specs=[
            pl.BlockSpec((1, 1, rb, d), lambda a, r: (a, 0, r, 0)),
            pl.BlockSpec((1, 1, rb, d), lambda a, r: (a, 1, r, 0)),
            bs,
            bs,
        ],
        out_specs=bs,
        out_shape=jax.ShapeDtypeStruct((2, n_pad, d), F32),
    )(p, p, g, dinv)


def _tc_head(pool, w1, b1r, w2p, b2p, b, b_pad, d):
    def body(p_ref, w1_ref, b1_ref, w2_ref, b2_ref, o_ref):
        p0 = p_ref[0, 0, :b, :] + p_ref[0, 1, :b, :]
        p1 = p_ref[1, 0, :b, :] + p_ref[1, 1, :b, :]
        z = jnp.concatenate([p0, p1], axis=1)
        z = jax.nn.relu(
            jnp.dot(z, w1_ref[...], preferred_element_type=F32) + b1_ref[...]
        )
        o_ref[...] = (
            jnp.dot(z, w2_ref[...], preferred_element_type=F32) + b2_ref[...]
        )

    return pl.pallas_call(
        body,
        out_shape=jax.ShapeDtypeStruct((b, d), F32),
    )(pool, w1, b1r, w2p, b2p)


def kernel(x, edge_index, batch, W0, b0, Wc, W1, b1, W2, b2):
    n, d = x.shape
    a_dim, _, e = edge_index.shape
    l_dim = Wc.shape[1]
    b = 64
    rb = 1024

    n_pad = -(-(n + 1) // (NS * ECH)) * (NS * ECH)
    epg = NC * NS * 8 * ECH
    e_pad = -(-e // epg) * epg
    chunks_per_tile = e_pad // (NC * NS * ECH)
    b_pad = 72

    x = x.astype(F32)
    xp = jnp.pad(x, ((0, n_pad - n), (0, 0)))
    ei = edge_index.astype(I32)
    ei = jnp.concatenate(
        [ei, jnp.full((a_dim, 2, e_pad - e), n, I32)], axis=2
    )
    src0 = ei[0, 0].reshape(-1, ECH)
    dst0 = ei[0, 1].reshape(-1, ECH)
    src1 = ei[1, 0].reshape(-1, ECH)
    dst1 = ei[1, 1].reshape(-1, ECH)
    batch_p = jnp.concatenate(
        [batch.astype(I32), jnp.full((n_pad - n,), b, I32)]
    ).reshape(-1, ECH)

    zrows = jnp.zeros((n_pad // NS, d), F32)
    ones_h = jnp.ones((ECH, d), F32)
    b0r = b0.reshape(1, d).astype(F32)
    b1r = b1.reshape(1, d).astype(F32)
    w2p = jnp.pad(W2.astype(F32), ((0, 0), (0, d - W2.shape[1])))
    b2p = jnp.broadcast_to(b2.reshape(1, 1).astype(F32), (1, d))

    h = _tc_h(xp, W0.astype(F32), b0r, n_pad, d, rb)
    deg = _make_sc_deg(n_pad, d, chunks_per_tile)(dst0, dst1, ones_h, zrows)
    dinv = _tc_dinv(deg, n_pad, d, rb)

    sc_conv = _make_sc_conv(n_pad, d, chunks_per_tile)
    g = _tc_g1(h, Wc[:, 0].astype(F32), dinv, n_pad, d, rb)
    for layer in range(1, l_dim + 1):
        p = sc_conv(g, src0, dst0, src1, dst1, zrows)
        if layer < l_dim:
            g = _tc_step(p, g, dinv, Wc[:, layer].astype(F32), n_pad, d, rb)
        else:
            h3 = _tc_final(p, g, dinv, n_pad, d, rb)

    pool = _make_sc_pool(n_pad, d, b_pad)(h3, batch_p, zrows)
    z = _tc_head(pool, W1.astype(F32), b1r, w2p, b2p, b, b_pad, d)
    return z[:, : W2.shape[1]]
